# initial kernel scaffold (unmeasured)
import jax
import jax.numpy as jnp
from jax import lax
from jax.experimental import pallas as pl
from jax.experimental.pallas import tpu as pltpu


def kernel(x, dy):
    k, d = x.shape
    k2, f = dy.shape
    assert k == k2
    half = d // 2

    def body(x_ref, dy_ref, out_ref, sendbuf, recvbuf, send_sem, recv_sem):
        my_x = lax.axis_index("x")
        my_y = lax.axis_index("y")
        my_z = lax.axis_index("z")
        partner = (my_x, my_y, 1 - my_z)

        barrier = pltpu.get_barrier_semaphore()
        pl.semaphore_signal(
            barrier, inc=1,
            device_id=partner, device_id_type=pl.DeviceIdType.MESH,
        )
        pl.semaphore_wait(barrier, 1)

        xb = x_ref[...].astype(jnp.bfloat16)
        dyb = dy_ref[...].astype(jnp.bfloat16)
        p = lax.dot_general(
            xb, dyb, (((0,), (0,)), ((), ())),
            preferred_element_type=jnp.float32,
        )

        sendbuf[...] = lax.dynamic_slice_in_dim(
            p, (1 - my_z) * half, half, axis=0
        ).astype(jnp.bfloat16)
        rdma = pltpu.make_async_remote_copy(
            src_ref=sendbuf,
            dst_ref=recvbuf,
            send_sem=send_sem,
            recv_sem=recv_sem,
            device_id=partner,
            device_id_type=pl.DeviceIdType.MESH,
        )
        rdma.start()
        rdma.wait()

        keep = lax.dynamic_slice_in_dim(p, my_z * half, half, axis=0)
        out_ref[...] = keep + recvbuf[...].astype(jnp.float32)

    return pl.pallas_call(
        body,
        out_shape=jax.ShapeDtypeStruct((half, f), jnp.float32),
        in_specs=[
            pl.BlockSpec(memory_space=pltpu.VMEM),
            pl.BlockSpec(memory_space=pltpu.VMEM),
        ],
        out_specs=pl.BlockSpec(memory_space=pltpu.VMEM),
        scratch_shapes=[
            pltpu.VMEM((half, f), jnp.bfloat16),
            pltpu.VMEM((half, f), jnp.bfloat16),
            pltpu.SemaphoreType.DMA,
            pltpu.SemaphoreType.DMA,
        ],
        compiler_params=pltpu.CompilerParams(collective_id=0),
    )(x, dy)


# baseline (device time: 76349 ns/iter reference)
import jax
import jax.numpy as jnp
from jax import lax
from jax.experimental import pallas as pl
from jax.experimental.pallas import tpu as pltpu


def kernel(x, dy):
    k, d = x.shape
    k2, f = dy.shape
    assert k == k2
    half = d // 2

    def body(x_ref, dy_ref, out_ref, p_ref, sendbuf, recvbuf, send_sem, recv_sem):
        my_x = lax.axis_index("x")
        my_y = lax.axis_index("y")
        my_z = lax.axis_index("z")
        partner = (my_x, my_y, 1 - my_z)

        barrier = pltpu.get_barrier_semaphore()
        pl.semaphore_signal(
            barrier, inc=1,
            device_id=partner, device_id_type=pl.DeviceIdType.MESH,
        )
        pl.semaphore_wait(barrier, 1)

        xb = x_ref[...].astype(jnp.bfloat16)
        dyb = dy_ref[...].astype(jnp.bfloat16)
        p_ref[...] = lax.dot_general(
            xb, dyb, (((0,), (0,)), ((), ())),
            preferred_element_type=jnp.float32,
        )

        sendbuf[...] = p_ref[pl.ds((1 - my_z) * half, half), :].astype(
            jnp.bfloat16
        )
        rdma = pltpu.make_async_remote_copy(
            src_ref=sendbuf,
            dst_ref=recvbuf,
            send_sem=send_sem,
            recv_sem=recv_sem,
            device_id=partner,
            device_id_type=pl.DeviceIdType.MESH,
        )
        rdma.start()
        rdma.wait()

        keep = p_ref[pl.ds(my_z * half, half), :]
        out_ref[...] = keep + recvbuf[...].astype(jnp.float32)

    return pl.pallas_call(
        body,
        out_shape=jax.ShapeDtypeStruct((half, f), jnp.float32),
        in_specs=[
            pl.BlockSpec(memory_space=pltpu.VMEM),
            pl.BlockSpec(memory_space=pltpu.VMEM),
        ],
        out_specs=pl.BlockSpec(memory_space=pltpu.VMEM),
        scratch_shapes=[
            pltpu.VMEM((d, f), jnp.float32),
            pltpu.VMEM((half, f), jnp.bfloat16),
            pltpu.VMEM((half, f), jnp.bfloat16),
            pltpu.SemaphoreType.DMA,
            pltpu.SemaphoreType.DMA,
        ],
        compiler_params=pltpu.CompilerParams(
            collective_id=0,
            vmem_limit_bytes=100 * 1024 * 1024,
        ),
    )(x, dy)


# device time: 48961 ns/iter; 1.5594x vs baseline; 1.5594x over previous
import jax
import jax.numpy as jnp
from jax import lax
from jax.experimental import pallas as pl
from jax.experimental.pallas import tpu as pltpu

NC = 8


def kernel(x, dy):
    k, d = x.shape
    k2, f = dy.shape
    assert k == k2
    half = d // 2
    piece = f // 4
    cc = piece // NC

    def body(x_ref, dy_ref, out_ref,
             kf, zsend, zrecv, wb, s1buf, s2buf,
             zs_sem, zr_sem, s1s_sem, s1r_sem, s2s_sem, s2r_sem):
        mx = lax.axis_index("x")
        my = lax.axis_index("y")
        mz = lax.axis_index("z")
        q = 2 * mx + my
        qx = 2 * (1 - mx) + my
        qy = 2 * mx + (1 - my)
        qd = 2 * (1 - mx) + (1 - my)

        zdev = (mx, my, 1 - mz)
        xdev = (1 - mx, my, mz)
        ydev = (mx, 1 - my, mz)

        barrier = pltpu.get_barrier_semaphore()
        for nbr in (zdev, xdev, ydev):
            pl.semaphore_signal(
                barrier, inc=1,
                device_id=nbr, device_id_type=pl.DeviceIdType.MESH,
            )
        pl.semaphore_wait(barrier, 3)

        xk = x_ref[:, pl.ds(mz * half, half)].astype(jnp.bfloat16)
        xs = x_ref[:, pl.ds((1 - mz) * half, half)].astype(jnp.bfloat16)

        def z_rdma(c):
            return pltpu.make_async_remote_copy(
                src_ref=zsend.at[c], dst_ref=zrecv.at[c],
                send_sem=zs_sem.at[c], recv_sem=zr_sem.at[c],
                device_id=zdev, device_id_type=pl.DeviceIdType.MESH,
            )

        def s1_rdma(c):
            return pltpu.make_async_remote_copy(
                src_ref=wb.at[c], dst_ref=s1buf.at[c],
                send_sem=s1s_sem.at[c], recv_sem=s1r_sem.at[c],
                device_id=(xdev if c % 2 == 0 else ydev),
                device_id_type=pl.DeviceIdType.MESH,
            )

        def s2_rdma(c, j):
            return pltpu.make_async_remote_copy(
                src_ref=(wb.at[c] if j == 0 else s1buf.at[c]),
                dst_ref=s2buf.at[c, j],
                send_sem=s2s_sem.at[c, j], recv_sem=s2r_sem.at[c, j],
                device_id=(ydev if c % 2 == 0 else xdev),
                device_id_type=pl.DeviceIdType.MESH,
            )

        for c in range(NC):
            dyc = dy_ref[:, pl.ds(q * piece + c * cc, cc)].astype(jnp.bfloat16)
            kf[c] = lax.dot_general(
                xk, dyc, (((0,), (0,)), ((), ())),
                preferred_element_type=jnp.float32,
            )
            zsend[c] = lax.dot_general(
                xs, dyc, (((0,), (0,)), ((), ())),
                preferred_element_type=jnp.float32,
            ).astype(jnp.bfloat16)
            z_rdma(c).start()

        for c in range(NC):
            z_rdma(c).wait()
            w = kf[c] + zrecv[c].astype(jnp.float32)
            out_ref[:, pl.ds(q * piece + c * cc, cc)] = w
            wb[c] = w.astype(jnp.bfloat16)
            s1_rdma(c).start()

        for c in range(NC):
            s1_rdma(c).wait()
            q1 = qx if c % 2 == 0 else qy
            out_ref[:, pl.ds(q1 * piece + c * cc, cc)] = (
                s1buf[c].astype(jnp.float32))
            s2_rdma(c, 0).start()
            s2_rdma(c, 1).start()

        for c in range(NC):
            s2_rdma(c, 0).wait()
            s2_rdma(c, 1).wait()
            qa = qy if c % 2 == 0 else qx
            out_ref[:, pl.ds(qa * piece + c * cc, cc)] = (
                s2buf[c, 0].astype(jnp.float32))
            out_ref[:, pl.ds(qd * piece + c * cc, cc)] = (
                s2buf[c, 1].astype(jnp.float32))

    return pl.pallas_call(
        body,
        out_shape=jax.ShapeDtypeStruct((half, f), jnp.float32),
        in_specs=[
            pl.BlockSpec(memory_space=pltpu.VMEM),
            pl.BlockSpec(memory_space=pltpu.VMEM),
        ],
        out_specs=pl.BlockSpec(memory_space=pltpu.VMEM),
        scratch_shapes=[
            pltpu.VMEM((NC, half, cc), jnp.float32),
            pltpu.VMEM((NC, half, cc), jnp.bfloat16),
            pltpu.VMEM((NC, half, cc), jnp.bfloat16),
            pltpu.VMEM((NC, half, cc), jnp.bfloat16),
            pltpu.VMEM((NC, half, cc), jnp.bfloat16),
            pltpu.VMEM((NC, 2, half, cc), jnp.bfloat16),
            pltpu.SemaphoreType.DMA((NC,)),
            pltpu.SemaphoreType.DMA((NC,)),
            pltpu.SemaphoreType.DMA((NC,)),
            pltpu.SemaphoreType.DMA((NC,)),
            pltpu.SemaphoreType.DMA((NC, 2)),
            pltpu.SemaphoreType.DMA((NC, 2)),
        ],
        compiler_params=pltpu.CompilerParams(
            collective_id=0,
            vmem_limit_bytes=100 * 1024 * 1024,
        ),
    )(x, dy)
